# baseline (speedup 1.0000x reference)
import jax
import jax.numpy as jnp
from jax import lax
from jax.experimental import pallas as pl
from jax.experimental.pallas import tpu as pltpu
from jax.experimental.pallas import tpu_sc as plsc

NC, NS = 2, 16
NW = NC * NS
B, P, D = 1024, 77, 768
ROWS_PER_W = B * P // NW
R = 8
NSTEP = ROWS_PER_W // R
NB = 4
PE = P + R - 1
LANES = 16
G = D // LANES


def _body(idx_hbm, table_hbm, pos_hbm, out_hbm, dump_hbm,
          idx_v, pos_v, bufs, gsems, wsems):
    wid = lax.axis_index("s") * NC + lax.axis_index("c")
    base = wid * ROWS_PER_W

    pltpu.sync_copy(idx_hbm.at[pl.ds(base, ROWS_PER_W + NB * R)], idx_v)
    pltpu.sync_copy(pos_hbm, pos_v)

    def gather(s, b):
        pltpu.async_copy(
            table_hbm.at[idx_v.at[pl.ds(s * R, R)]], bufs.at[b], gsems.at[b]
        )

    def gwait(b):
        pltpu.make_async_copy(
            table_hbm.at[idx_v.at[pl.ds(0, R)]], bufs.at[b], gsems.at[b]
        ).wait()

    def add_pos(s, b):
        p0d = lax.rem(s * R, P) * D
        K = 8
        for r in range(R):
            for g0 in range(0, G, K):
                vs = [
                    pos_v[pl.ds(p0d + r * D + (g0 + k) * LANES, LANES)]
                    for k in range(K)
                ]
                for k in range(K):
                    plsc.addupdate(
                        bufs.at[b, r, pl.ds((g0 + k) * LANES, LANES)], vs[k]
                    )

    def wstart(s, b):
        pltpu.async_copy(
            bufs.at[b], out_hbm.at[(base + s * R) // 8], wsems.at[b]
        )

    def wwait(b):
        pltpu.make_async_copy(bufs.at[b], out_hbm.at[0], wsems.at[b]).wait()

    gather(0, 0)
    gather(1, 1)
    gather(2, 2)
    pltpu.async_copy(bufs.at[3], dump_hbm, wsems.at[3])

    def quad(gi, c):
        s0 = 4 * gi
        for ss in range(4):
            s = s0 + ss
            gwait(ss)
            wwait((ss + 3) % NB)
            gather(s + 3, (ss + 3) % NB)
            add_pos(s, ss)
            wstart(s, ss)
        return c

    lax.fori_loop(0, NSTEP // 4, quad, 0)

    gwait(0)
    gwait(1)
    gwait(2)
    wwait(3)


def kernel(input_tokens, token_table, position_table):
    idx = jnp.pad(input_tokens.astype(jnp.int32).reshape(-1), (0, NB * R))
    pos = jnp.concatenate([position_table, position_table[: R - 1]]).reshape(-1)
    mesh = plsc.VectorSubcoreMesh(
        core_axis_name="c", subcore_axis_name="s", num_cores=NC, num_subcores=NS
    )
    out, _ = pl.kernel(
        _body,
        out_type=(
            jax.ShapeDtypeStruct((B * P // 8, 8, D), jnp.float32),
            jax.ShapeDtypeStruct((8, D), jnp.float32),
        ),
        mesh=mesh,
        scratch_types=[
            pltpu.VMEM((ROWS_PER_W + NB * R,), jnp.int32),
            pltpu.VMEM((PE * D,), jnp.float32),
            pltpu.VMEM((NB, 8, D), jnp.float32),
            pltpu.SemaphoreType.DMA((NB,)),
            pltpu.SemaphoreType.DMA((NB,)),
        ],
    )(idx, token_table, pos)
    return out.reshape(B, P, D)
